# baseline (speedup 1.0000x reference)
import functools

import jax
import jax.numpy as jnp
from jax import lax
from jax.experimental import pallas as pl
from jax.experimental.pallas import tpu as pltpu
from jax.experimental.pallas import tpu_sc as plsc

B, L, V, D = 4096, 50, 1000000, 64
NC, NS = 2, 16
NW = NC * NS
BPW = B // NW
NPAIR = BPW * L
NGRP = NPAIR // 16
B2 = B + 16
VCB = 15
VC = 1 << VCB
NCHK = 31
LASTC_BASE = (NCHK - 1) * VC
LASTC_MAIN = 12928
TAILW = 4096
TAIL_DST = LASTC_MAIN - 64
EPAD = B * L + NW * NCHK * 16 + 16
TRASH = EPAD - 16
PSN = NPAIR + NCHK * 16
DUMMY_B = B
EB = 1024

_mesh = plsc.VectorSubcoreMesh(
    core_axis_name="c", subcore_axis_name="s", num_cores=NC, num_subcores=NS)
_params = pltpu.CompilerParams(
    use_tc_tiling_on_sc=True, needs_layout_passes=False)
_IOTA = functools.partial(jnp.arange, dtype=jnp.int32)


def _wid():
    return lax.axis_index("s") * NC + lax.axis_index("c")


def _load_idx_block(idxF, ib, sem, wid):
    base = pl.multiple_of(BPW * wid, BPW)
    for j in range(L):
        pltpu.async_copy(
            idxF.at[pl.ds(j * B + base, BPW)], ib.at[pl.ds(BPW * j, BPW)], sem)
    for j in range(L):
        pltpu.make_async_copy(
            idxF.at[pl.ds(j * B + base, BPW)], ib.at[pl.ds(BPW * j, BPW)],
            sem).wait()


def _hist_body(idxF, hist, ib, h, sem):
    wid = _wid()
    _load_idx_block(idxF, ib, sem, wid)
    z = jnp.zeros((16,), jnp.int32)
    for k in range(4):
        h[pl.ds(16 * k, 16)] = z

    def g_body(g, carry):
        v = ib[pl.ds(g * 16, 16)]
        c = v >> VCB
        cnt, last = plsc.scan_count(c)
        plsc.addupdate_scatter(h, [c], cnt, mask=last)
        return carry

    lax.fori_loop(0, NGRP, g_body, 0)
    pltpu.sync_copy(h, hist.at[pl.ds(pl.multiple_of(64 * wid, 64), 64)])


def _scal(mb, pos):
    vec = plsc.load_gather(mb, [jnp.full((16,), pos, jnp.int32)])
    return lax.reduce_max(vec, axes=(0,))


def _bucket_body(idxF, hist, entries, meta, ib, hb, ctr, fin, mv, ps, es, psp, esp, sem):
    wid = _wid()
    _load_idx_block(idxF, ib, sem, wid)
    pltpu.sync_copy(hist, hb)

    tots, bases = [], []
    run = jnp.int32(0)
    for k in range(4):
        t = jnp.zeros((16,), jnp.int32)
        for s in range(NW):
            t = t + ((hb[pl.ds(64 * s + 16 * k, 16)] + 15) & ~15)
        tots.append(t)
        bases.append(plsc.cumsum(t) - t + run)
        run = run + lax.reduce_sum(t, axes=(0,))

    def s_body(s, carry):
        return tuple(
            carry[k] + ((hb[pl.ds(64 * s + 16 * k, 16)] + 15) & ~15)
            for k in range(4))

    pref = lax.fori_loop(
        0, wid, s_body, tuple(jnp.zeros((16,), jnp.int32) for _ in range(4)))
    f_regs = []
    for k in range(4):
        start = bases[k] + pref[k]
        ctr[pl.ds(16 * k, 16)] = start
        own = hb[pl.ds(64 * wid + 16 * k, 16)]
        fin[pl.ds(16 * k, 16)] = start + own
        f_regs.append(start + own)

    @pl.when(wid == 0)
    def _():
        for k in range(4):
            mv[pl.ds(16 * k, 16)] = bases[k]
            mv[pl.ds(64 + 16 * k, 16)] = tots[k]
        pltpu.sync_copy(mv, meta)

    iota = _IOTA(16)

    def g_body(g, carry):
        v = ib[pl.ds(g * 16, 16)]
        c = v >> VCB
        loc = v - (c << VCB)
        bl = lax.rem(g, 8) * 16
        b = BPW * wid + bl + iota
        e = (b << VCB) | loc
        cnt, last = plsc.scan_count(c)
        cur = plsc.load_gather(ctr, [c])
        ps[pl.ds(g * 16, 16)] = cur + cnt - 1
        es[pl.ds(g * 16, 16)] = e
        plsc.addupdate_scatter(ctr, [c], cnt, mask=last)
        return carry

    lax.fori_loop(0, NGRP, g_body, 0)
    dummy = jnp.full((16,), DUMMY_B << VCB, jnp.int32)
    neg = jnp.full((16,), -2147483648, jnp.int32)
    for c in range(NCHK):
        st = lax.reduce_max(
            jnp.where(iota == (c % 16), f_regs[c // 16], neg), axes=(0,))
        pads = (-st) & 15
        pos = jnp.where(iota < pads, st + iota, TRASH + iota)
        psp[pl.ds(16 * c, 16)] = pos
        esp[pl.ds(16 * c, 16)] = dummy
    pltpu.sync_copy(es, entries.at[ps])
    pltpu.sync_copy(esp, entries.at[psp])


def _main_body(tabT, tail, entries, meta, outT, tb0, tb1, eb0, eb1, accb, res,
               mb, st0, st1, se0, se1):
    wid = _wid()
    pltpu.sync_copy(meta, mb)
    iota = _IOTA(16)
    bank_off = (iota & 7) * B2
    m_lo = iota < 8
    tbufs = (tb0, st0), (tb1, st1)

    def tab_pairs(c, d, tbuf):
        if c < NCHK - 1:
            return ((tabT.at[d, pl.ds(c * VC, VC)], tbuf),)
        tofs = pl.multiple_of(d * TAILW, TAILW)
        return ((tabT.at[d, pl.ds(LASTC_BASE, LASTC_MAIN)],
                 tbuf.at[pl.ds(0, LASTC_MAIN)]),
                (tail.at[pl.ds(tofs, TAILW)],
                 tbuf.at[pl.ds(TAIL_DST, TAILW)]))

    def pass_body(p, carry_p):
        d = 2 * wid + p

        def z_body(t, carry):
            z = jnp.zeros((16,), jnp.float32)
            for k in range(4):
                accb[pl.ds(t * 64 + 16 * k, 16)] = z
            return carry

        lax.fori_loop(0, 8 * B2 // 64, z_body, 0)

        for c in range(2):
            tbuf, st = tbufs[c % 2]
            for src, dst in tab_pairs(c, d, tbuf):
                pltpu.async_copy(src, dst, st)

        for c in range(NCHK):
            tbuf, st = tbufs[c % 2]
            for src, dst in tab_pairs(c, d, tbuf):
                pltpu.make_async_copy(src, dst, st).wait()
            base_e = (pl.multiple_of(_scal(mb, c), 16)
                      if c > 0 else jnp.int32(0))
            n_c = _scal(mb, 64 + c)
            nb = (n_c + EB - 1) >> 10

            @pl.when(nb > 0)
            def _():
                pltpu.async_copy(entries.at[pl.ds(base_e, EB)], eb0, se0)

            @pl.when(nb > 1)
            def _():
                pltpu.async_copy(entries.at[pl.ds(base_e + EB, EB)], eb1, se1)

            def blk2(t2, carry2):
                for h2, ebuf, se in ((0, eb0, se0), (1, eb1, se1)):
                    i = 2 * t2 + h2

                    @pl.when(i < nb)
                    def _():
                        pltpu.make_async_copy(
                            entries.at[pl.ds(base_e + i * EB, EB)],
                            ebuf, se).wait()

                        def g_body(g, carry3):
                            msk = (i * EB + g * 16 + iota) < n_c
                            e = ebuf[pl.ds(g * 16, 16)]
                            b = e >> VCB
                            loc = e & (VC - 1)
                            v = plsc.load_gather(tbuf, [loc], mask=msk)
                            bidx = b + bank_off
                            plsc.addupdate_scatter(
                                accb, [bidx], v, mask=msk & m_lo)
                            plsc.addupdate_scatter(
                                accb, [bidx], v, mask=msk & ~m_lo)
                            return carry3

                        lax.fori_loop(0, EB // 16, g_body, 0)

                        @pl.when(i + 2 < nb)
                        def _():
                            pltpu.async_copy(
                                entries.at[pl.ds(base_e + (i + 2) * EB, EB)],
                                ebuf, se)
                return carry2

            lax.fori_loop(0, (nb + 1) >> 1, blk2, 0)
            if c + 2 < NCHK:
                tb2, st2 = tbufs[c % 2]
                for src, dst in tab_pairs(c + 2, d, tb2):
                    pltpu.async_copy(src, dst, st2)

        def r_body(q, carry):
            s = accb[pl.ds(q * 16, 16)]
            for k in range(1, 8):
                s = s + accb[pl.ds(k * B2 + q * 16, 16)]
            res[pl.ds(q * 16, 16)] = s
            return carry

        lax.fori_loop(0, B // 16, r_body, 0)
        pltpu.sync_copy(res, outT.at[d])
        return carry_p

    lax.fori_loop(0, 2, pass_body, 0)


_p0a = pl.kernel(
    _hist_body,
    out_type=jax.ShapeDtypeStruct((NW * 64,), jnp.int32),
    mesh=_mesh,
    scratch_types=[
        pltpu.VMEM((NPAIR,), jnp.int32),
        pltpu.VMEM((64,), jnp.int32),
        pltpu.SemaphoreType.DMA,
    ],
    compiler_params=_params,
)

_p0b = pl.kernel(
    _bucket_body,
    out_type=(
        jax.ShapeDtypeStruct((EPAD,), jnp.int32),
        jax.ShapeDtypeStruct((128,), jnp.int32),
    ),
    mesh=_mesh,
    scratch_types=[
        pltpu.VMEM((NPAIR,), jnp.int32),
        pltpu.VMEM((64 * NW,), jnp.int32),
        pltpu.VMEM((64,), jnp.int32),
        pltpu.VMEM((64,), jnp.int32),
        pltpu.VMEM((128,), jnp.int32),
        pltpu.VMEM((NPAIR,), jnp.int32),
        pltpu.VMEM((NPAIR,), jnp.int32),
        pltpu.VMEM((NCHK * 16,), jnp.int32),
        pltpu.VMEM((NCHK * 16,), jnp.int32),
        pltpu.SemaphoreType.DMA,
    ],
    compiler_params=_params,
)

_pmain = pl.kernel(
    _main_body,
    out_type=jax.ShapeDtypeStruct((D, B), jnp.float32),
    mesh=_mesh,
    scratch_types=[
        pltpu.VMEM((VC,), jnp.float32),
        pltpu.VMEM((VC,), jnp.float32),
        pltpu.VMEM((EB,), jnp.int32),
        pltpu.VMEM((EB,), jnp.int32),
        pltpu.VMEM((8 * B2,), jnp.float32),
        pltpu.VMEM((B,), jnp.float32),
        pltpu.VMEM((128,), jnp.int32),
        pltpu.SemaphoreType.DMA,
        pltpu.SemaphoreType.DMA,
        pltpu.SemaphoreType.DMA,
        pltpu.SemaphoreType.DMA,
    ],
    compiler_params=_params,
)


def kernel(idx, table):
    idxF = idx.astype(jnp.int32).T.reshape(L * B)
    tabT = table.T
    tail = lax.slice_in_dim(tabT, V - TAILW, V, axis=1).reshape(D * TAILW)
    hist = _p0a(idxF)
    entries, meta = _p0b(idxF, hist)
    outT = _pmain(tabT, tail, entries, meta)
    return outT.T[:, None, :]


# speedup vs baseline: 2.8535x; 2.8535x over previous
import functools

import jax
import jax.numpy as jnp
from jax import lax
from jax.experimental import pallas as pl
from jax.experimental.pallas import tpu as pltpu
from jax.experimental.pallas import tpu_sc as plsc

B, L, V, D = 4096, 50, 1000000, 64
NC, NS = 2, 16
NW = NC * NS
BPW = B // NW
CB = 2
NCHUNK = BPW // CB
CIDX = CB * L
LANES = 16


NBUF = 4
KCH = 4


def _sc_kernel(idx_hbm, table_hbm, out_hbm, idx_v, bufs, out_v, *sems):
    wid = lax.axis_index("s") * NC + lax.axis_index("c")
    pltpu.sync_copy(idx_hbm.at[pl.ds(wid * NCHUNK, NCHUNK)], idx_v)

    def issue(c, b):
        pltpu.async_copy(table_hbm.at[idx_v.at[c]], bufs.at[b], sems[b])

    for b in range(NBUF):
        issue(b, b)

    def group_body(g, carry):
        for b in range(NBUF):
            c = g * NBUF + b
            pltpu.make_async_copy(
                table_hbm.at[idx_v.at[c]], bufs.at[b], sems[b]).wait()
            buf = bufs.at[b]
            for ro in range(CB):
                base = ro * L
                for q in range(D // LANES):
                    ds = pl.ds(q * LANES, LANES)
                    accs = [None] * KCH
                    for j in range(L):
                        v = buf[base + j, ds]
                        k = j % KCH
                        accs[k] = v if accs[k] is None else accs[k] + v
                    while len(accs) > 1:
                        accs = [a + bb for a, bb in zip(accs[::2], accs[1::2])] \
                            + ([accs[-1]] if len(accs) % 2 else [])
                    out_v[c * CB + ro, ds] = accs[0]
            nxt = c + NBUF

            @pl.when(nxt < NCHUNK)
            def _():
                issue(nxt, b)
        return carry

    lax.fori_loop(0, NCHUNK // NBUF, group_body, 0)
    pltpu.sync_copy(out_v, out_hbm.at[pl.ds(wid * BPW, BPW)])


@jax.jit
def _run(idx2d, table):
    mesh = plsc.VectorSubcoreMesh(
        core_axis_name="c", subcore_axis_name="s",
        num_cores=NC, num_subcores=NS)
    return pl.kernel(
        _sc_kernel,
        out_type=jax.ShapeDtypeStruct((B, D), jnp.float32),
        mesh=mesh,
        scratch_types=[
            pltpu.VMEM((NCHUNK, CIDX), jnp.int32),
            pltpu.VMEM((NBUF, CIDX, D), jnp.float32),
            pltpu.VMEM((BPW, D), jnp.float32),
        ] + [pltpu.SemaphoreType.DMA] * NBUF,
        compiler_params=pltpu.CompilerParams(use_tc_tiling_on_sc=False),
    )(idx2d, table)


def kernel(idx, table):
    idx2d = idx.astype(jnp.int32).reshape(NW * NCHUNK, CIDX)
    out = _run(idx2d, table)
    return out[:, None, :]
